# baseline (speedup 1.0000x reference)
import functools

import jax
import jax.numpy as jnp
from jax import lax
from jax.experimental import pallas as pl
from jax.experimental.pallas import tpu as pltpu
from jax.experimental.pallas import tpu_sc as plsc

NC = 2
NS = 16
NW = NC * NS
CHUNK = 128


def kernel(table0, table1, table2, table3, indexes):
    B = indexes.shape[0]
    OUT_D = 128
    bpw = B // NW
    nch = bpw // CHUNK

    idxr = indexes.astype(jnp.int32).reshape(NW, nch, CHUNK)
    padded = [jnp.pad(t, ((0, 0), (0, OUT_D - t.shape[1])))
              for t in (table0, table1, table2, table3)]

    mesh = plsc.VectorSubcoreMesh(core_axis_name="c", subcore_axis_name="s")

    @functools.partial(
        pl.kernel,
        mesh=mesh,
        out_type=jax.ShapeDtypeStruct((B, OUT_D), jnp.float32),
        compiler_params=pltpu.CompilerParams(
            use_tc_tiling_on_sc=False, needs_layout_passes=False),
        scratch_types=[
            pltpu.VMEM((nch, CHUNK), jnp.int32),
            pltpu.VMEM((CHUNK, OUT_D), jnp.float32),
            pltpu.VMEM((CHUNK, OUT_D), jnp.float32),
            pltpu.VMEM((CHUNK, OUT_D), jnp.float32),
            pltpu.VMEM((CHUNK, OUT_D), jnp.float32),
            pltpu.SemaphoreType.DMA((4,)),
        ],
    )
    def sc_kernel(t0h, t1h, t2h, t3h, idx_hbm, out_hbm,
                  idx_v, g0, g1, g2, g3, sem):
        wid = lax.axis_index("s") * NC + lax.axis_index("c")
        base = wid * bpw
        pltpu.sync_copy(idx_hbm.at[wid], idx_v)
        for j in range(nch):
            ij = idx_v.at[j]
            cps = [pltpu.async_copy(th.at[ij], g, sem.at[k])
                   for k, (th, g) in enumerate(
                       ((t0h, g0), (t1h, g1), (t2h, g2), (t3h, g3)))]
            orows = pl.ds(base + j * CHUNK, CHUNK)
            for k, (c, g) in enumerate(zip(cps, (g0, g1, g2, g3))):
                c.wait()
                pltpu.sync_copy(g.at[:, pl.ds(0, 32)],
                                out_hbm.at[orows, pl.ds(32 * k, 32)])

    return sc_kernel(*padded, idxr)
